# baseline (speedup 1.0000x reference)
import functools

import jax
import jax.numpy as jnp
from jax import lax
from jax.experimental import pallas as pl
from jax.experimental.pallas import tpu as pltpu
from jax.experimental.pallas import tpu_sc as plsc

K = 16
B = 1024
D = 16
N = 100000
C = 2048
NCH = (N + C - 1) // C
NP = NCH * C
BR = 1024
NR = B // BR
SB = C // 128

NW = 32
QW = B // NW
IW = QW * K


def _topk_body(trainT_ref, test_ref, tr_ref, te_ref, idx_ref, tv_ref, ti_ref):
    c = pl.program_id(1)

    @pl.when(c == 0)
    def _init():
        tv_ref[:] = jnp.full((BR, K), 0x7F800000, jnp.int32)
        ti_ref[:] = jnp.zeros((BR, K), jnp.int32)

    xc = trainT_ref[:]
    t = test_ref[:]
    tr = tr_ref[0]
    te = te_ref[:]
    mm = jnp.dot(t, xc, preferred_element_type=jnp.float32)
    d = jnp.sqrt(jnp.maximum(te + tr - 2.0 * mm, 0.0))
    di0 = lax.bitcast_convert_type(d, jnp.int32)

    BIGI = jnp.int32(0x7FFFFFFF)
    lane128 = lax.broadcasted_iota(jnp.int32, (BR, 128), 1)
    j16 = lax.broadcasted_iota(jnp.int32, (BR, K), 1)
    jsb = lax.broadcasted_iota(jnp.int32, (BR, SB), 1)

    def subscan(mf, pf, use_floor):
        ms_l, ps_l = [], []
        for s in range(SB):
            ds = di0[:, s * 128:(s + 1) * 128]
            if use_floor:
                live = (ds > mf[:, s:s + 1]) | (
                    (ds == mf[:, s:s + 1]) & (lane128 > pf[:, s:s + 1]))
                ds = jnp.where(live, ds, BIGI)
            msv = jnp.min(ds, axis=1, keepdims=True)
            psv = jnp.min(jnp.where(ds == msv, lane128, 128), axis=1,
                          keepdims=True)
            ms_l.append(msv)
            ps_l.append(psv)
        return jnp.concatenate(ms_l, 1), jnp.concatenate(ps_l, 1)

    def run_inner(msub, pos, tv, ti, mf, pf):
        m0 = jnp.min(msub, axis=1, keepdims=True)

        def icond(st):
            msub, pos, tv, ti, mf, pf, m, cnt = st
            return jnp.logical_and(cnt < K, jnp.any(m < tv[:, K - 1:K]))

        def ibody(st):
            msub, pos, tv, ti, mf, pf, m, cnt = st
            sb = jnp.min(jnp.where(msub == m, jsb, SB), axis=1, keepdims=True)
            issb = jsb == sb
            l = jnp.sum(jnp.where(issb, pos, 0), axis=1, keepdims=True)
            g = sb * 128 + l + c * C
            posn = jnp.sum((tv <= m).astype(jnp.int32), axis=1, keepdims=True)
            tv_sh = jnp.concatenate([m, tv[:, :K - 1]], axis=1)
            ti_sh = jnp.concatenate([g, ti[:, :K - 1]], axis=1)
            keep = j16 < posn
            ins = j16 == posn
            tv = jnp.where(keep, tv, jnp.where(ins, jnp.broadcast_to(m, (BR, K)), tv_sh))
            ti = jnp.where(keep, ti, jnp.where(ins, jnp.broadcast_to(g, (BR, K)), ti_sh))
            msub = jnp.where(issb, BIGI, msub)
            mf = jnp.where(issb, jnp.broadcast_to(m, (BR, SB)), mf)
            pf = jnp.where(issb, jnp.broadcast_to(l, (BR, SB)), pf)
            m2 = jnp.min(msub, axis=1, keepdims=True)
            return msub, pos, tv, ti, mf, pf, m2, cnt + 1

        return lax.while_loop(
            icond, ibody, (msub, pos, tv, ti, mf, pf, m0, jnp.int32(0)))

    tv = tv_ref[:]
    ti = ti_ref[:]
    mf0 = jnp.full((BR, SB), -1, jnp.int32)
    pf0 = jnp.full((BR, SB), -1, jnp.int32)
    msub, pos = subscan(mf0, pf0, False)
    _, _, tv, ti, mf, pf, _, cnt = run_inner(msub, pos, tv, ti, mf0, pf0)

    def ocond(st):
        tv, ti, mf, pf, cnt, ito = st
        return jnp.logical_and(cnt > 0, ito < K + 2)

    def obody(st):
        tv, ti, mf, pf, cnt, ito = st
        msub, pos = subscan(mf, pf, True)
        _, _, tv, ti, mf, pf, _, cnt2 = run_inner(msub, pos, tv, ti, mf, pf)
        return tv, ti, mf, pf, cnt2, ito + 1

    tv, ti, _, _, _, _ = lax.while_loop(
        ocond, obody, (tv, ti, mf, pf, cnt, jnp.int32(0)))
    tv_ref[:] = tv
    ti_ref[:] = ti

    @pl.when(c == NCH - 1)
    def _out():
        idx_ref[:] = ti


def _topk_call(trainT_p, test_x, tr_pad, te):
    return pl.pallas_call(
        _topk_body,
        grid=(NR, NCH),
        in_specs=[
            pl.BlockSpec((D, C), lambda r, c: (0, c)),
            pl.BlockSpec((BR, D), lambda r, c: (r, 0)),
            pl.BlockSpec((1, 1, C), lambda r, c: (c, 0, 0)),
            pl.BlockSpec((BR, 1), lambda r, c: (r, 0)),
        ],
        out_specs=pl.BlockSpec((BR, K), lambda r, c: (r, 0)),
        out_shape=jax.ShapeDtypeStruct((B, K), jnp.int32),
        scratch_shapes=[
            pltpu.VMEM((BR, K), jnp.int32),
            pltpu.VMEM((BR, K), jnp.int32),
        ],
    )(trainT_p, test_x, tr_pad, te)


def _gather_mean(train_y, idx_flat):
    mesh = plsc.VectorSubcoreMesh(core_axis_name="c", subcore_axis_name="s")

    @functools.partial(
        pl.kernel,
        mesh=mesh,
        out_type=jax.ShapeDtypeStruct((B, D), jnp.float32),
        scratch_types=[
            pltpu.VMEM((IW,), jnp.int32),
            pltpu.VMEM((IW, D), jnp.float32),
            pltpu.VMEM((QW, D), jnp.float32),
            pltpu.SemaphoreType.DMA,
        ],
        compiler_params=pltpu.CompilerParams(use_tc_tiling_on_sc=False),
    )
    def k(y_hbm, idx_hbm, out_hbm, idx_v, rows_v, acc_v, sem):
        wid = lax.axis_index("s") * 2 + lax.axis_index("c")
        base = wid * IW
        pltpu.sync_copy(idx_hbm.at[pl.ds(base, IW)], idx_v)
        pltpu.async_copy(y_hbm.at[idx_v], rows_v, sem).wait()

        def q_body(q, carry):
            acc = rows_v[q * K]
            for j in range(1, K):
                acc = acc + rows_v[q * K + j]
            acc_v[q] = acc * (1.0 / K)
            return carry

        lax.fori_loop(0, QW, q_body, 0)
        pltpu.sync_copy(acc_v, out_hbm.at[pl.ds(wid * QW, QW)])

    return k(train_y, idx_flat)


def kernel(train_x, train_y, test_x):
    trainT = jnp.pad(train_x, ((0, NP - N), (0, 0))).T
    tr = jnp.sum(train_x ** 2, axis=1, keepdims=True)
    te = jnp.sum(test_x ** 2, axis=1, keepdims=True)
    tr_pad = jnp.pad(tr[:, 0], (0, NP - N),
                     constant_values=jnp.inf).reshape(NCH, 1, C)
    idx = _topk_call(trainT, test_x, tr_pad, te)
    return _gather_mean(train_y, idx.reshape(B * K))


# speedup vs baseline: 3.0946x; 3.0946x over previous
import functools

import jax
import jax.numpy as jnp
from jax import lax
from jax.experimental import pallas as pl
from jax.experimental.pallas import tpu as pltpu
from jax.experimental.pallas import tpu_sc as plsc

K = 16
B = 1024
D = 16
N = 100000
C = 2048
NCH = (N + C - 1) // C
NP = NCH * C
SB = C // 128

NW = 32
QW = B // NW
IW = QW * K


def _topk_body(train_ref, testT_ref, tr_ref, teT_ref, idx_ref, tv_ref, ti_ref):
    c = pl.program_id(0)

    @pl.when(c == 0)
    def _init():
        tv_ref[:] = jnp.full((K, B), 0x7F800000, jnp.int32)
        ti_ref[:] = jnp.zeros((K, B), jnp.int32)

    xc = train_ref[:]
    tT = testT_ref[:]
    trc = jnp.reshape(tr_ref[0], (C, 1))
    te = teT_ref[:]
    mm = jnp.dot(xc, tT, preferred_element_type=jnp.float32)
    d = jnp.sqrt(jnp.maximum(te + trc - 2.0 * mm, 0.0))
    di3 = jnp.reshape(lax.bitcast_convert_type(d, jnp.int32), (SB, 128, B))

    BIGI = jnp.int32(0x7FFFFFFF)
    slane = lax.broadcasted_iota(jnp.int32, (SB, 128, B), 1)
    jk = lax.broadcasted_iota(jnp.int32, (K, B), 0)
    jsb = lax.broadcasted_iota(jnp.int32, (SB, B), 0)

    def subscan(mf, pf, use_floor):
        dl = di3
        if use_floor:
            live = (di3 > mf[:, None, :]) | (
                (di3 == mf[:, None, :]) & (slane > pf[:, None, :]))
            dl = jnp.where(live, di3, BIGI)
        msub = jnp.min(dl, axis=1)
        psub = jnp.min(jnp.where(dl == msub[:, None, :], slane, 128), axis=1)
        return msub, psub

    def run_inner(msub, psub, tv, ti, mf, pf):
        m0 = jnp.min(msub, axis=0, keepdims=True)

        def icond(st):
            msub, psub, tv, ti, mf, pf, m, cnt = st
            return jnp.logical_and(cnt < K, jnp.any(m < tv[K - 1:K, :]))

        def ibody(st):
            msub, psub, tv, ti, mf, pf, m, cnt = st
            sb = jnp.min(jnp.where(msub == m, jsb, SB), axis=0, keepdims=True)
            issb = jsb == sb
            l = jnp.sum(jnp.where(issb, psub, 0), axis=0, keepdims=True)
            g = sb * 128 + l + c * C
            posn = jnp.sum((tv <= m).astype(jnp.int32), axis=0, keepdims=True)
            tv_sh = jnp.concatenate([m, tv[:K - 1, :]], axis=0)
            ti_sh = jnp.concatenate([g, ti[:K - 1, :]], axis=0)
            keep = jk < posn
            ins = jk == posn
            tv = jnp.where(keep, tv, jnp.where(ins, jnp.broadcast_to(m, (K, B)), tv_sh))
            ti = jnp.where(keep, ti, jnp.where(ins, jnp.broadcast_to(g, (K, B)), ti_sh))
            msub = jnp.where(issb, BIGI, msub)
            mf = jnp.where(issb, jnp.broadcast_to(m, (SB, B)), mf)
            pf = jnp.where(issb, jnp.broadcast_to(l, (SB, B)), pf)
            m2 = jnp.min(msub, axis=0, keepdims=True)
            return msub, psub, tv, ti, mf, pf, m2, cnt + 1

        return lax.while_loop(
            icond, ibody, (msub, psub, tv, ti, mf, pf, m0, jnp.int32(0)))

    tv = tv_ref[:]
    ti = ti_ref[:]
    mf0 = jnp.full((SB, B), -1, jnp.int32)
    pf0 = jnp.full((SB, B), -1, jnp.int32)
    msub, psub = subscan(mf0, pf0, False)
    _, _, tv, ti, mf, pf, _, cnt = run_inner(msub, psub, tv, ti, mf0, pf0)

    def ocond(st):
        tv, ti, mf, pf, cnt, ito = st
        return jnp.logical_and(cnt > 0, ito < K + 2)

    def obody(st):
        tv, ti, mf, pf, cnt, ito = st
        msub, psub = subscan(mf, pf, True)
        _, _, tv, ti, mf, pf, _, cnt2 = run_inner(msub, psub, tv, ti, mf, pf)
        return tv, ti, mf, pf, cnt2, ito + 1

    tv, ti, _, _, _, _ = lax.while_loop(
        ocond, obody, (tv, ti, mf, pf, cnt, jnp.int32(0)))
    tv_ref[:] = tv
    ti_ref[:] = ti

    @pl.when(c == NCH - 1)
    def _out():
        idx_ref[:] = ti


def _topk_call(train_pad, testT, tr_pad, teT):
    return pl.pallas_call(
        _topk_body,
        grid=(NCH,),
        in_specs=[
            pl.BlockSpec((C, D), lambda c: (c, 0)),
            pl.BlockSpec((D, B), lambda c: (0, 0)),
            pl.BlockSpec((1, 1, C), lambda c: (c, 0, 0)),
            pl.BlockSpec((1, B), lambda c: (0, 0)),
        ],
        out_specs=pl.BlockSpec((K, B), lambda c: (0, 0)),
        out_shape=jax.ShapeDtypeStruct((K, B), jnp.int32),
        scratch_shapes=[
            pltpu.VMEM((K, B), jnp.int32),
            pltpu.VMEM((K, B), jnp.int32),
        ],
    )(train_pad, testT, tr_pad, teT)


def _gather_mean(train_y, idx_flat):
    mesh = plsc.VectorSubcoreMesh(core_axis_name="c", subcore_axis_name="s")

    @functools.partial(
        pl.kernel,
        mesh=mesh,
        out_type=jax.ShapeDtypeStruct((B, D), jnp.float32),
        scratch_types=[
            pltpu.VMEM((IW,), jnp.int32),
            pltpu.VMEM((IW, D), jnp.float32),
            pltpu.VMEM((QW, D), jnp.float32),
            pltpu.SemaphoreType.DMA,
        ],
        compiler_params=pltpu.CompilerParams(use_tc_tiling_on_sc=False),
    )
    def k(y_hbm, idx_hbm, out_hbm, idx_v, rows_v, acc_v, sem):
        wid = lax.axis_index("s") * 2 + lax.axis_index("c")
        base = wid * IW
        pltpu.sync_copy(idx_hbm.at[pl.ds(base, IW)], idx_v)
        pltpu.async_copy(y_hbm.at[idx_v], rows_v, sem).wait()

        def q_body(q, carry):
            acc = rows_v[q * K]
            for j in range(1, K):
                acc = acc + rows_v[q * K + j]
            acc_v[q] = acc * (1.0 / K)
            return carry

        lax.fori_loop(0, QW, q_body, 0)
        pltpu.sync_copy(acc_v, out_hbm.at[pl.ds(wid * QW, QW)])

    return k(train_y, idx_flat)


def kernel(train_x, train_y, test_x):
    train_pad = jnp.pad(train_x, ((0, NP - N), (0, 0)))
    testT = test_x.T
    tr = jnp.sum(train_x ** 2, axis=1, keepdims=True)
    te = jnp.sum(test_x ** 2, axis=1, keepdims=True)
    tr_pad = jnp.pad(tr[:, 0], (0, NP - N),
                     constant_values=jnp.inf).reshape(NCH, 1, C)
    idx = _topk_call(train_pad, testT, tr_pad, te.T)
    return _gather_mean(train_y, idx.T.reshape(B * K))


# speedup vs baseline: 3.1065x; 1.0039x over previous
import functools

import jax
import jax.numpy as jnp
from jax import lax
from jax.experimental import pallas as pl
from jax.experimental.pallas import tpu as pltpu
from jax.experimental.pallas import tpu_sc as plsc

K = 16
B = 1024
D = 16
N = 100000
C = 4096
NCH = (N + C - 1) // C
NP = NCH * C
SB = C // 128

NW = 32
QW = B // NW
IW = QW * K


def _topk_body(train_ref, testT_ref, tr_ref, teT_ref, idx_ref, tv_ref, ti_ref):
    c = pl.program_id(0)

    @pl.when(c == 0)
    def _init():
        tv_ref[:] = jnp.full((K, B), 0x7F800000, jnp.int32)
        ti_ref[:] = jnp.zeros((K, B), jnp.int32)

    xc = train_ref[:]
    tT = testT_ref[:]
    trc = jnp.reshape(tr_ref[0], (C, 1))
    te = teT_ref[:]
    mm = jnp.dot(xc, tT, preferred_element_type=jnp.float32)
    d = jnp.sqrt(jnp.maximum(te + trc - 2.0 * mm, 0.0))
    di3 = jnp.reshape(lax.bitcast_convert_type(d, jnp.int32), (SB, 128, B))

    BIGI = jnp.int32(0x7FFFFFFF)
    slane = lax.broadcasted_iota(jnp.int32, (SB, 128, B), 1)
    jk = lax.broadcasted_iota(jnp.int32, (K, B), 0)
    jsb = lax.broadcasted_iota(jnp.int32, (SB, B), 0)

    def subscan(mf, pf, use_floor):
        dl = di3
        if use_floor:
            live = (di3 > mf[:, None, :]) | (
                (di3 == mf[:, None, :]) & (slane > pf[:, None, :]))
            dl = jnp.where(live, di3, BIGI)
        msub = jnp.min(dl, axis=1)
        psub = jnp.min(jnp.where(dl == msub[:, None, :], slane, 128), axis=1)
        return msub, psub

    def run_inner(msub, psub, tv, ti, mf, pf):
        m0 = jnp.min(msub, axis=0, keepdims=True)

        def icond(st):
            msub, psub, tv, ti, mf, pf, m, cnt = st
            return jnp.logical_and(cnt < K, jnp.any(m < tv[K - 1:K, :]))

        def ibody(st):
            msub, psub, tv, ti, mf, pf, m, cnt = st
            sb = jnp.min(jnp.where(msub == m, jsb, SB), axis=0, keepdims=True)
            issb = jsb == sb
            l = jnp.sum(jnp.where(issb, psub, 0), axis=0, keepdims=True)
            g = sb * 128 + l + c * C
            posn = jnp.sum((tv <= m).astype(jnp.int32), axis=0, keepdims=True)
            tv_sh = jnp.concatenate([m, tv[:K - 1, :]], axis=0)
            ti_sh = jnp.concatenate([g, ti[:K - 1, :]], axis=0)
            keep = jk < posn
            ins = jk == posn
            tv = jnp.where(keep, tv, jnp.where(ins, jnp.broadcast_to(m, (K, B)), tv_sh))
            ti = jnp.where(keep, ti, jnp.where(ins, jnp.broadcast_to(g, (K, B)), ti_sh))
            msub = jnp.where(issb, BIGI, msub)
            mf = jnp.where(issb, jnp.broadcast_to(m, (SB, B)), mf)
            pf = jnp.where(issb, jnp.broadcast_to(l, (SB, B)), pf)
            m2 = jnp.min(msub, axis=0, keepdims=True)
            return msub, psub, tv, ti, mf, pf, m2, cnt + 1

        return lax.while_loop(
            icond, ibody, (msub, psub, tv, ti, mf, pf, m0, jnp.int32(0)))

    tv = tv_ref[:]
    ti = ti_ref[:]
    mf0 = jnp.full((SB, B), -1, jnp.int32)
    pf0 = jnp.full((SB, B), -1, jnp.int32)
    msub, psub = subscan(mf0, pf0, False)
    _, _, tv, ti, mf, pf, _, cnt = run_inner(msub, psub, tv, ti, mf0, pf0)

    def ocond(st):
        tv, ti, mf, pf, cnt, ito = st
        return jnp.logical_and(cnt > 0, ito < K + 2)

    def obody(st):
        tv, ti, mf, pf, cnt, ito = st
        msub, psub = subscan(mf, pf, True)
        _, _, tv, ti, mf, pf, _, cnt2 = run_inner(msub, psub, tv, ti, mf, pf)
        return tv, ti, mf, pf, cnt2, ito + 1

    tv, ti, _, _, _, _ = lax.while_loop(
        ocond, obody, (tv, ti, mf, pf, cnt, jnp.int32(0)))
    tv_ref[:] = tv
    ti_ref[:] = ti

    @pl.when(c == NCH - 1)
    def _out():
        idx_ref[:] = ti


def _topk_call(train_pad, testT, tr_pad, teT):
    return pl.pallas_call(
        _topk_body,
        grid=(NCH,),
        in_specs=[
            pl.BlockSpec((C, D), lambda c: (c, 0)),
            pl.BlockSpec((D, B), lambda c: (0, 0)),
            pl.BlockSpec((1, 1, C), lambda c: (c, 0, 0)),
            pl.BlockSpec((1, B), lambda c: (0, 0)),
        ],
        out_specs=pl.BlockSpec((K, B), lambda c: (0, 0)),
        out_shape=jax.ShapeDtypeStruct((K, B), jnp.int32),
        scratch_shapes=[
            pltpu.VMEM((K, B), jnp.int32),
            pltpu.VMEM((K, B), jnp.int32),
        ],
    )(train_pad, testT, tr_pad, teT)


def _gather_mean(train_y, idx_flat):
    mesh = plsc.VectorSubcoreMesh(core_axis_name="c", subcore_axis_name="s")

    @functools.partial(
        pl.kernel,
        mesh=mesh,
        out_type=jax.ShapeDtypeStruct((B, D), jnp.float32),
        scratch_types=[
            pltpu.VMEM((IW,), jnp.int32),
            pltpu.VMEM((IW, D), jnp.float32),
            pltpu.VMEM((QW, D), jnp.float32),
            pltpu.SemaphoreType.DMA,
        ],
        compiler_params=pltpu.CompilerParams(use_tc_tiling_on_sc=False),
    )
    def k(y_hbm, idx_hbm, out_hbm, idx_v, rows_v, acc_v, sem):
        wid = lax.axis_index("s") * 2 + lax.axis_index("c")
        base = wid * IW
        pltpu.sync_copy(idx_hbm.at[pl.ds(base, IW)], idx_v)
        pltpu.async_copy(y_hbm.at[idx_v], rows_v, sem).wait()

        def q_body(q, carry):
            acc = rows_v[q * K]
            for j in range(1, K):
                acc = acc + rows_v[q * K + j]
            acc_v[q] = acc * (1.0 / K)
            return carry

        lax.fori_loop(0, QW, q_body, 0)
        pltpu.sync_copy(acc_v, out_hbm.at[pl.ds(wid * QW, QW)])

    return k(train_y, idx_flat)


def kernel(train_x, train_y, test_x):
    train_pad = jnp.pad(train_x, ((0, NP - N), (0, 0)))
    testT = test_x.T
    tr = jnp.sum(train_x ** 2, axis=1, keepdims=True)
    te = jnp.sum(test_x ** 2, axis=1, keepdims=True)
    tr_pad = jnp.pad(tr[:, 0], (0, NP - N),
                     constant_values=jnp.inf).reshape(NCH, 1, C)
    idx = _topk_call(train_pad, testT, tr_pad, te.T)
    return _gather_mean(train_y, idx.T.reshape(B * K))


# speedup vs baseline: 3.1420x; 1.0114x over previous
import functools

import jax
import jax.numpy as jnp
from jax import lax
from jax.experimental import pallas as pl
from jax.experimental.pallas import tpu as pltpu
from jax.experimental.pallas import tpu_sc as plsc

K = 16
B = 1024
D = 16
N = 100000
C = 2048
NCH = (N + C - 1) // C
NP = NCH * C
SB = C // 128

NW = 32
QW = B // NW
IW = QW * K


def _topk_body(train_ref, testT_ref, tr_ref, teT_ref, idx_ref, tv_ref, ti_ref):
    c = pl.program_id(0)

    @pl.when(c == 0)
    def _init():
        tv_ref[:] = jnp.full((K, B), 0x7F800000, jnp.int32)
        ti_ref[:] = jnp.zeros((K, B), jnp.int32)

    xc = train_ref[:]
    tT = testT_ref[:]
    trc = jnp.reshape(tr_ref[0], (C, 1))
    te = teT_ref[:]
    mm = jnp.dot(xc, tT, preferred_element_type=jnp.float32)
    d = jnp.sqrt(jnp.maximum(te + trc - 2.0 * mm, 0.0))
    di3 = jnp.reshape(lax.bitcast_convert_type(d, jnp.int32), (SB, 128, B))

    BIGI = jnp.int32(0x7FFFFFFF)
    slane = lax.broadcasted_iota(jnp.int32, (SB, 128, B), 1)
    jk = lax.broadcasted_iota(jnp.int32, (K, B), 0)
    jsb = lax.broadcasted_iota(jnp.int32, (SB, B), 0)

    def subscan(mf, pf, use_floor):
        dl = di3
        if use_floor:
            live = (di3 > mf[:, None, :]) | (
                (di3 == mf[:, None, :]) & (slane > pf[:, None, :]))
            dl = jnp.where(live, di3, BIGI)
        msub = jnp.min(dl, axis=1)
        psub = jnp.min(jnp.where(dl == msub[:, None, :], slane, 128), axis=1)
        return msub, psub

    def run_inner(msub, psub, tv, ti, mf, pf):
        m0 = jnp.min(msub, axis=0, keepdims=True)

        def icond(st):
            msub, psub, tv, ti, mf, pf, m, cnt = st
            return jnp.logical_and(cnt < K, jnp.any(m < tv[K - 1:K, :]))

        def ibody(st):
            msub, psub, tv, ti, mf, pf, m, cnt = st
            sb = jnp.min(jnp.where(msub == m, jsb, SB), axis=0, keepdims=True)
            issb = jsb == sb
            l = jnp.sum(jnp.where(issb, psub, 0), axis=0, keepdims=True)
            g = sb * 128 + l + c * C
            posn = jnp.sum((tv <= m).astype(jnp.int32), axis=0, keepdims=True)
            tv_sh = jnp.concatenate([m, tv[:K - 1, :]], axis=0)
            ti_sh = jnp.concatenate([g, ti[:K - 1, :]], axis=0)
            keep = jk < posn
            ins = jk == posn
            tv = jnp.where(keep, tv, jnp.where(ins, jnp.broadcast_to(m, (K, B)), tv_sh))
            ti = jnp.where(keep, ti, jnp.where(ins, jnp.broadcast_to(g, (K, B)), ti_sh))
            msub = jnp.where(issb, BIGI, msub)
            mf = jnp.where(issb, jnp.broadcast_to(m, (SB, B)), mf)
            pf = jnp.where(issb, jnp.broadcast_to(l, (SB, B)), pf)
            m2 = jnp.min(msub, axis=0, keepdims=True)
            return msub, psub, tv, ti, mf, pf, m2, cnt + 1

        return lax.while_loop(
            icond, ibody, (msub, psub, tv, ti, mf, pf, m0, jnp.int32(0)))

    def run_inner_cached(msub, psub, msub2, psub2, tv, ti, mf, pf):
        m0 = jnp.min(msub, axis=0, keepdims=True)

        def icond(st):
            msub, psub, msub2, psub2, tv, ti, mf, pf, m, cnt = st
            return jnp.logical_and(cnt < K, jnp.any(m < tv[K - 1:K, :]))

        def ibody(st):
            msub, psub, msub2, psub2, tv, ti, mf, pf, m, cnt = st
            sb = jnp.min(jnp.where(msub == m, jsb, SB), axis=0, keepdims=True)
            issb = jsb == sb
            l = jnp.sum(jnp.where(issb, psub, 0), axis=0, keepdims=True)
            g = sb * 128 + l + c * C
            posn = jnp.sum((tv <= m).astype(jnp.int32), axis=0, keepdims=True)
            tv_sh = jnp.concatenate([m, tv[:K - 1, :]], axis=0)
            ti_sh = jnp.concatenate([g, ti[:K - 1, :]], axis=0)
            keep = jk < posn
            ins = jk == posn
            tv = jnp.where(keep, tv, jnp.where(ins, jnp.broadcast_to(m, (K, B)), tv_sh))
            ti = jnp.where(keep, ti, jnp.where(ins, jnp.broadcast_to(g, (K, B)), ti_sh))
            msub = jnp.where(issb, msub2, msub)
            psub = jnp.where(issb, psub2, psub)
            msub2 = jnp.where(issb, BIGI, msub2)
            mf = jnp.where(issb, jnp.broadcast_to(m, (SB, B)), mf)
            pf = jnp.where(issb, jnp.broadcast_to(l, (SB, B)), pf)
            m2 = jnp.min(msub, axis=0, keepdims=True)
            return msub, psub, msub2, psub2, tv, ti, mf, pf, m2, cnt + 1

        return lax.while_loop(
            icond, ibody,
            (msub, psub, msub2, psub2, tv, ti, mf, pf, m0, jnp.int32(0)))

    tv = tv_ref[:]
    ti = ti_ref[:]
    mf0 = jnp.full((SB, B), -1, jnp.int32)
    pf0 = jnp.full((SB, B), -1, jnp.int32)
    msub, psub = subscan(mf0, pf0, False)
    m0 = jnp.min(msub, axis=0, keepdims=True)
    need = jnp.any(m0 < tv[K - 1:K, :])

    def _mk2():
        live = (di3 > msub[:, None, :]) | (
            (di3 == msub[:, None, :]) & (slane > psub[:, None, :]))
        dl2 = jnp.where(live, di3, BIGI)
        ms2 = jnp.min(dl2, axis=1)
        ps2 = jnp.min(jnp.where(dl2 == ms2[:, None, :], slane, 128), axis=1)
        return ms2, ps2

    def _mk2_dummy():
        return (jnp.full((SB, B), BIGI, jnp.int32),
                jnp.full((SB, B), 128, jnp.int32))

    msub2, psub2 = lax.cond(need, _mk2, _mk2_dummy)
    msub_f, _, _, _, tv, ti, mf, pf, _, _ = run_inner_cached(
        msub, psub, msub2, psub2, tv, ti, mf0, pf0)
    go0 = jnp.any((msub_f == BIGI) &
                  (mf < jnp.broadcast_to(tv[K - 1:K, :], (SB, B))))

    def ocond(st):
        tv, ti, mf, pf, go, ito = st
        return jnp.logical_and(go, ito < 2 * K)

    def obody(st):
        tv, ti, mf, pf, go, ito = st
        msub, psub = subscan(mf, pf, True)
        _, _, tv, ti, mf, pf, _, cnt2 = run_inner(msub, psub, tv, ti, mf, pf)
        return tv, ti, mf, pf, cnt2 > 0, ito + 1

    tv, ti, _, _, _, _ = lax.while_loop(
        ocond, obody, (tv, ti, mf, pf, go0, jnp.int32(0)))
    tv_ref[:] = tv
    ti_ref[:] = ti

    @pl.when(c == NCH - 1)
    def _out():
        idx_ref[:] = ti


def _topk_call(train_pad, testT, tr_pad, teT):
    return pl.pallas_call(
        _topk_body,
        grid=(NCH,),
        in_specs=[
            pl.BlockSpec((C, D), lambda c: (c, 0)),
            pl.BlockSpec((D, B), lambda c: (0, 0)),
            pl.BlockSpec((1, 1, C), lambda c: (c, 0, 0)),
            pl.BlockSpec((1, B), lambda c: (0, 0)),
        ],
        out_specs=pl.BlockSpec((K, B), lambda c: (0, 0)),
        out_shape=jax.ShapeDtypeStruct((K, B), jnp.int32),
        scratch_shapes=[
            pltpu.VMEM((K, B), jnp.int32),
            pltpu.VMEM((K, B), jnp.int32),
        ],
    )(train_pad, testT, tr_pad, teT)


def _gather_mean(train_y, idx_flat):
    mesh = plsc.VectorSubcoreMesh(core_axis_name="c", subcore_axis_name="s")

    @functools.partial(
        pl.kernel,
        mesh=mesh,
        out_type=jax.ShapeDtypeStruct((B, D), jnp.float32),
        scratch_types=[
            pltpu.VMEM((IW,), jnp.int32),
            pltpu.VMEM((IW, D), jnp.float32),
            pltpu.VMEM((QW, D), jnp.float32),
            pltpu.SemaphoreType.DMA,
        ],
        compiler_params=pltpu.CompilerParams(use_tc_tiling_on_sc=False),
    )
    def k(y_hbm, idx_hbm, out_hbm, idx_v, rows_v, acc_v, sem):
        wid = lax.axis_index("s") * 2 + lax.axis_index("c")
        base = wid * IW
        pltpu.sync_copy(idx_hbm.at[pl.ds(base, IW)], idx_v)
        pltpu.async_copy(y_hbm.at[idx_v], rows_v, sem).wait()

        def q_body(q, carry):
            acc = rows_v[q * K]
            for j in range(1, K):
                acc = acc + rows_v[q * K + j]
            acc_v[q] = acc * (1.0 / K)
            return carry

        lax.fori_loop(0, QW, q_body, 0)
        pltpu.sync_copy(acc_v, out_hbm.at[pl.ds(wid * QW, QW)])

    return k(train_y, idx_flat)


def kernel(train_x, train_y, test_x):
    train_pad = jnp.pad(train_x, ((0, NP - N), (0, 0)))
    testT = test_x.T
    tr = jnp.sum(train_x ** 2, axis=1, keepdims=True)
    te = jnp.sum(test_x ** 2, axis=1, keepdims=True)
    tr_pad = jnp.pad(tr[:, 0], (0, NP - N),
                     constant_values=jnp.inf).reshape(NCH, 1, C)
    idx = _topk_call(train_pad, testT, tr_pad, te.T)
    return _gather_mean(train_y, idx.T.reshape(B * K))


# speedup vs baseline: 3.1800x; 1.0121x over previous
import functools

import jax
import jax.numpy as jnp
from jax import lax
from jax.experimental import pallas as pl
from jax.experimental.pallas import tpu as pltpu
from jax.experimental.pallas import tpu_sc as plsc

K = 16
B = 1024
D = 16
N = 100000
C = 2048
NCH = (N + C - 1) // C
NP = NCH * C
SB = C // 128

NW = 32
QW = B // NW
IW = QW * K


def _topk_body(train_ref, testT_ref, tr_ref, teT_ref, idx_ref, tv_ref, ti_ref):
    c = pl.program_id(0)

    @pl.when(c == 0)
    def _init():
        tv_ref[:] = jnp.full((K, B), 0x7F800000, jnp.int32)
        ti_ref[:] = jnp.zeros((K, B), jnp.int32)

    xc = train_ref[:]
    tT = testT_ref[:]
    trc = jnp.reshape(tr_ref[0], (C, 1))
    te = teT_ref[:]
    mm = jnp.dot(xc, tT, preferred_element_type=jnp.float32)
    d = jnp.sqrt(jnp.maximum(te + trc - 2.0 * mm, 0.0))
    di3 = jnp.reshape(lax.bitcast_convert_type(d, jnp.int32), (SB, 128, B))

    BIGI = jnp.int32(0x7FFFFFFF)
    slane = lax.broadcasted_iota(jnp.int32, (SB, 128, B), 1)
    jk = lax.broadcasted_iota(jnp.int32, (K, B), 0)
    jsb = lax.broadcasted_iota(jnp.int32, (SB, B), 0)

    def subscan(mf, pf, use_floor):
        dl = di3
        if use_floor:
            live = (di3 > mf[:, None, :]) | (
                (di3 == mf[:, None, :]) & (slane > pf[:, None, :]))
            dl = jnp.where(live, di3, BIGI)
        msub = jnp.min(dl, axis=1)
        psub = jnp.min(jnp.where(dl == msub[:, None, :], slane, 128), axis=1)
        return msub, psub

    def run_inner(msub, psub, tv, ti, mf, pf):
        m0 = jnp.min(msub, axis=0, keepdims=True)

        def icond(st):
            msub, psub, tv, ti, mf, pf, m, cnt = st
            return jnp.logical_and(cnt < K, jnp.any(m < tv[K - 1:K, :]))

        def ibody(st):
            msub, psub, tv, ti, mf, pf, m, cnt = st
            sb = jnp.min(jnp.where(msub == m, jsb, SB), axis=0, keepdims=True)
            issb = jsb == sb
            l = jnp.sum(jnp.where(issb, psub, 0), axis=0, keepdims=True)
            g = sb * 128 + l + c * C
            posn = jnp.sum((tv <= m).astype(jnp.int32), axis=0, keepdims=True)
            tv_sh = jnp.concatenate([m, tv[:K - 1, :]], axis=0)
            ti_sh = jnp.concatenate([g, ti[:K - 1, :]], axis=0)
            keep = jk < posn
            ins = jk == posn
            tv = jnp.where(keep, tv, jnp.where(ins, jnp.broadcast_to(m, (K, B)), tv_sh))
            ti = jnp.where(keep, ti, jnp.where(ins, jnp.broadcast_to(g, (K, B)), ti_sh))
            msub = jnp.where(issb, BIGI, msub)
            mf = jnp.where(issb, jnp.broadcast_to(m, (SB, B)), mf)
            pf = jnp.where(issb, jnp.broadcast_to(l, (SB, B)), pf)
            m2 = jnp.min(msub, axis=0, keepdims=True)
            return msub, psub, tv, ti, mf, pf, m2, cnt + 1

        return lax.while_loop(
            icond, lambda st: ibody(ibody(st)),
            (msub, psub, tv, ti, mf, pf, m0, jnp.int32(0)))

    def run_inner_cached(msub, psub, msub2, psub2, tv, ti, mf, pf):
        m0 = jnp.min(msub, axis=0, keepdims=True)

        def icond(st):
            msub, psub, msub2, psub2, tv, ti, mf, pf, m, cnt = st
            return jnp.logical_and(cnt < K, jnp.any(m < tv[K - 1:K, :]))

        def ibody(st):
            msub, psub, msub2, psub2, tv, ti, mf, pf, m, cnt = st
            sb = jnp.min(jnp.where(msub == m, jsb, SB), axis=0, keepdims=True)
            issb = jsb == sb
            l = jnp.sum(jnp.where(issb, psub, 0), axis=0, keepdims=True)
            g = sb * 128 + l + c * C
            posn = jnp.sum((tv <= m).astype(jnp.int32), axis=0, keepdims=True)
            tv_sh = jnp.concatenate([m, tv[:K - 1, :]], axis=0)
            ti_sh = jnp.concatenate([g, ti[:K - 1, :]], axis=0)
            keep = jk < posn
            ins = jk == posn
            tv = jnp.where(keep, tv, jnp.where(ins, jnp.broadcast_to(m, (K, B)), tv_sh))
            ti = jnp.where(keep, ti, jnp.where(ins, jnp.broadcast_to(g, (K, B)), ti_sh))
            msub = jnp.where(issb, msub2, msub)
            psub = jnp.where(issb, psub2, psub)
            msub2 = jnp.where(issb, BIGI, msub2)
            mf = jnp.where(issb, jnp.broadcast_to(m, (SB, B)), mf)
            pf = jnp.where(issb, jnp.broadcast_to(l, (SB, B)), pf)
            m2 = jnp.min(msub, axis=0, keepdims=True)
            return msub, psub, msub2, psub2, tv, ti, mf, pf, m2, cnt + 1

        return lax.while_loop(
            icond, lambda st: ibody(ibody(st)),
            (msub, psub, msub2, psub2, tv, ti, mf, pf, m0, jnp.int32(0)))

    tv = tv_ref[:]
    ti = ti_ref[:]
    mf0 = jnp.full((SB, B), -1, jnp.int32)
    pf0 = jnp.full((SB, B), -1, jnp.int32)
    msub, psub = subscan(mf0, pf0, False)
    m0 = jnp.min(msub, axis=0, keepdims=True)
    need = jnp.any(m0 < tv[K - 1:K, :])

    def _mk2():
        live = (di3 > msub[:, None, :]) | (
            (di3 == msub[:, None, :]) & (slane > psub[:, None, :]))
        dl2 = jnp.where(live, di3, BIGI)
        ms2 = jnp.min(dl2, axis=1)
        ps2 = jnp.min(jnp.where(dl2 == ms2[:, None, :], slane, 128), axis=1)
        return ms2, ps2

    def _mk2_dummy():
        return (jnp.full((SB, B), BIGI, jnp.int32),
                jnp.full((SB, B), 128, jnp.int32))

    msub2, psub2 = lax.cond(need, _mk2, _mk2_dummy)
    msub_f, _, _, _, tv, ti, mf, pf, _, _ = run_inner_cached(
        msub, psub, msub2, psub2, tv, ti, mf0, pf0)
    go0 = jnp.any((msub_f == BIGI) &
                  (mf < jnp.broadcast_to(tv[K - 1:K, :], (SB, B))))

    def ocond(st):
        tv, ti, mf, pf, go, ito = st
        return jnp.logical_and(go, ito < 2 * K)

    def obody(st):
        tv, ti, mf, pf, go, ito = st
        msub, psub = subscan(mf, pf, True)
        _, _, tv, ti, mf, pf, _, cnt2 = run_inner(msub, psub, tv, ti, mf, pf)
        return tv, ti, mf, pf, cnt2 > 0, ito + 1

    tv, ti, _, _, _, _ = lax.while_loop(
        ocond, obody, (tv, ti, mf, pf, go0, jnp.int32(0)))
    tv_ref[:] = tv
    ti_ref[:] = ti

    @pl.when(c == NCH - 1)
    def _out():
        idx_ref[:] = ti


def _topk_call(train_pad, testT, tr_pad, teT):
    return pl.pallas_call(
        _topk_body,
        grid=(NCH,),
        in_specs=[
            pl.BlockSpec((C, D), lambda c: (c, 0)),
            pl.BlockSpec((D, B), lambda c: (0, 0)),
            pl.BlockSpec((1, 1, C), lambda c: (c, 0, 0)),
            pl.BlockSpec((1, B), lambda c: (0, 0)),
        ],
        out_specs=pl.BlockSpec((K, B), lambda c: (0, 0)),
        out_shape=jax.ShapeDtypeStruct((K, B), jnp.int32),
        scratch_shapes=[
            pltpu.VMEM((K, B), jnp.int32),
            pltpu.VMEM((K, B), jnp.int32),
        ],
    )(train_pad, testT, tr_pad, teT)


def _gather_mean(train_y, idx_flat):
    mesh = plsc.VectorSubcoreMesh(core_axis_name="c", subcore_axis_name="s")

    @functools.partial(
        pl.kernel,
        mesh=mesh,
        out_type=jax.ShapeDtypeStruct((B, D), jnp.float32),
        scratch_types=[
            pltpu.VMEM((IW,), jnp.int32),
            pltpu.VMEM((IW, D), jnp.float32),
            pltpu.VMEM((QW, D), jnp.float32),
            pltpu.SemaphoreType.DMA,
        ],
        compiler_params=pltpu.CompilerParams(use_tc_tiling_on_sc=False),
    )
    def k(y_hbm, idx_hbm, out_hbm, idx_v, rows_v, acc_v, sem):
        wid = lax.axis_index("s") * 2 + lax.axis_index("c")
        base = wid * IW
        pltpu.sync_copy(idx_hbm.at[pl.ds(base, IW)], idx_v)
        pltpu.async_copy(y_hbm.at[idx_v], rows_v, sem).wait()

        def q_body(q, carry):
            acc = rows_v[q * K]
            for j in range(1, K):
                acc = acc + rows_v[q * K + j]
            acc_v[q] = acc * (1.0 / K)
            return carry

        lax.fori_loop(0, QW, q_body, 0)
        pltpu.sync_copy(acc_v, out_hbm.at[pl.ds(wid * QW, QW)])

    return k(train_y, idx_flat)


def kernel(train_x, train_y, test_x):
    train_pad = jnp.pad(train_x, ((0, NP - N), (0, 0)))
    testT = test_x.T
    tr = jnp.sum(train_x ** 2, axis=1, keepdims=True)
    te = jnp.sum(test_x ** 2, axis=1, keepdims=True)
    tr_pad = jnp.pad(tr[:, 0], (0, NP - N),
                     constant_values=jnp.inf).reshape(NCH, 1, C)
    idx = _topk_call(train_pad, testT, tr_pad, te.T)
    return _gather_mean(train_y, idx.T.reshape(B * K))


# speedup vs baseline: 3.1817x; 1.0005x over previous
import functools

import jax
import jax.numpy as jnp
from jax import lax
from jax.experimental import pallas as pl
from jax.experimental.pallas import tpu as pltpu
from jax.experimental.pallas import tpu_sc as plsc

K = 16
B = 1024
D = 16
N = 100000
C = 2048
NCH = (N + C - 1) // C
NP = NCH * C
SB = C // 128

NW = 32
QW = B // NW
IW = QW * K


def _topk_body(train_ref, testT_ref, tr_ref, teT_ref, idx_ref, tv_ref, ti_ref):
    c = pl.program_id(0)

    @pl.when(c == 0)
    def _init():
        tv_ref[:] = jnp.full((K, B), 0x7F800000, jnp.int32)
        ti_ref[:] = lax.broadcasted_iota(jnp.int32, (K, B), 0)

    xc = train_ref[:]
    tT = testT_ref[:]
    trc = jnp.reshape(tr_ref[0], (C, 1))
    te = teT_ref[:]
    mm = jnp.dot(xc, tT, preferred_element_type=jnp.float32)
    d = jnp.sqrt(jnp.maximum(te + trc - 2.0 * mm, 0.0))
    di3 = jnp.reshape(lax.bitcast_convert_type(d, jnp.int32), (SB, 128, B))

    BIGI = jnp.int32(0x7FFFFFFF)
    slane = lax.broadcasted_iota(jnp.int32, (SB, 128, B), 1)
    jk = lax.broadcasted_iota(jnp.int32, (K, B), 0)
    jsb = lax.broadcasted_iota(jnp.int32, (SB, B), 0)

    def subscan(mf, pf, use_floor):
        dl = di3
        if use_floor:
            live = (di3 > mf[:, None, :]) | (
                (di3 == mf[:, None, :]) & (slane > pf[:, None, :]))
            dl = jnp.where(live, di3, BIGI)
        msub = jnp.min(dl, axis=1)
        psub = jnp.min(jnp.where(dl == msub[:, None, :], slane, 128), axis=1)
        return msub, psub

    def run_inner(msub, psub, tv, ti, mf, pf, w):
        m0 = jnp.min(msub, axis=0, keepdims=True)

        def icond(st):
            msub, psub, tv, ti, mf, pf, m, w, cnt = st
            return jnp.logical_and(cnt < K, jnp.any(m < w))

        def ibody(st):
            msub, psub, tv, ti, mf, pf, m, w, cnt = st
            sb = jnp.min(jnp.where(msub == m, jsb, SB), axis=0, keepdims=True)
            issb = jsb == sb
            l = jnp.sum(jnp.where(issb, psub, 0), axis=0, keepdims=True)
            g = sb * 128 + l + c * C
            isw = tv == w
            gmax = jnp.max(jnp.where(isw, ti, -1), axis=0, keepdims=True)
            repl = isw & (ti == gmax) & (m < w)
            tv = jnp.where(repl, jnp.broadcast_to(m, (K, B)), tv)
            ti = jnp.where(repl, jnp.broadcast_to(g, (K, B)), ti)
            w2 = jnp.max(tv, axis=0, keepdims=True)
            msub = jnp.where(issb, BIGI, msub)
            mf = jnp.where(issb, jnp.broadcast_to(m, (SB, B)), mf)
            pf = jnp.where(issb, jnp.broadcast_to(l, (SB, B)), pf)
            m2 = jnp.min(msub, axis=0, keepdims=True)
            return msub, psub, tv, ti, mf, pf, m2, w2, cnt + 1

        return lax.while_loop(
            icond, lambda st: ibody(ibody(st)),
            (msub, psub, tv, ti, mf, pf, m0, w, jnp.int32(0)))

    def run_inner_cached(msub, psub, msub2, psub2, tv, ti, mf, pf, w):
        m0 = jnp.min(msub, axis=0, keepdims=True)

        def icond(st):
            msub, psub, msub2, psub2, tv, ti, mf, pf, m, w, cnt = st
            return jnp.logical_and(cnt < K, jnp.any(m < w))

        def ibody(st):
            msub, psub, msub2, psub2, tv, ti, mf, pf, m, w, cnt = st
            sb = jnp.min(jnp.where(msub == m, jsb, SB), axis=0, keepdims=True)
            issb = jsb == sb
            l = jnp.sum(jnp.where(issb, psub, 0), axis=0, keepdims=True)
            g = sb * 128 + l + c * C
            isw = tv == w
            gmax = jnp.max(jnp.where(isw, ti, -1), axis=0, keepdims=True)
            repl = isw & (ti == gmax) & (m < w)
            tv = jnp.where(repl, jnp.broadcast_to(m, (K, B)), tv)
            ti = jnp.where(repl, jnp.broadcast_to(g, (K, B)), ti)
            w2 = jnp.max(tv, axis=0, keepdims=True)
            msub = jnp.where(issb, msub2, msub)
            psub = jnp.where(issb, psub2, psub)
            msub2 = jnp.where(issb, BIGI, msub2)
            mf = jnp.where(issb, jnp.broadcast_to(m, (SB, B)), mf)
            pf = jnp.where(issb, jnp.broadcast_to(l, (SB, B)), pf)
            m2 = jnp.min(msub, axis=0, keepdims=True)
            return msub, psub, msub2, psub2, tv, ti, mf, pf, m2, w2, cnt + 1

        return lax.while_loop(
            icond, lambda st: ibody(ibody(st)),
            (msub, psub, msub2, psub2, tv, ti, mf, pf, m0, w, jnp.int32(0)))

    tv = tv_ref[:]
    ti = ti_ref[:]
    mf0 = jnp.full((SB, B), -1, jnp.int32)
    pf0 = jnp.full((SB, B), -1, jnp.int32)
    msub, psub = subscan(mf0, pf0, False)
    w = jnp.max(tv, axis=0, keepdims=True)
    m0 = jnp.min(msub, axis=0, keepdims=True)
    need = jnp.any(m0 < w)

    def _mk2():
        live = (di3 > msub[:, None, :]) | (
            (di3 == msub[:, None, :]) & (slane > psub[:, None, :]))
        dl2 = jnp.where(live, di3, BIGI)
        ms2 = jnp.min(dl2, axis=1)
        ps2 = jnp.min(jnp.where(dl2 == ms2[:, None, :], slane, 128), axis=1)
        return ms2, ps2

    def _mk2_dummy():
        return (jnp.full((SB, B), BIGI, jnp.int32),
                jnp.full((SB, B), 128, jnp.int32))

    msub2, psub2 = lax.cond(need, _mk2, _mk2_dummy)
    msub_f, _, _, _, tv, ti, mf, pf, _, w, _ = run_inner_cached(
        msub, psub, msub2, psub2, tv, ti, mf0, pf0, w)
    go0 = jnp.any((msub_f == BIGI) & (mf < jnp.broadcast_to(w, (SB, B))))

    def ocond(st):
        tv, ti, mf, pf, w, go, ito = st
        return jnp.logical_and(go, ito < 2 * K)

    def obody(st):
        tv, ti, mf, pf, w, go, ito = st
        msub, psub = subscan(mf, pf, True)
        _, _, tv, ti, mf, pf, _, w, cnt2 = run_inner(msub, psub, tv, ti, mf, pf, w)
        return tv, ti, mf, pf, w, cnt2 > 0, ito + 1

    tv, ti, _, _, _, _, _ = lax.while_loop(
        ocond, obody, (tv, ti, mf, pf, w, go0, jnp.int32(0)))
    tv_ref[:] = tv
    ti_ref[:] = ti

    @pl.when(c == NCH - 1)
    def _out():
        idx_ref[:] = ti


def _topk_call(train_pad, testT, tr_pad, teT):
    return pl.pallas_call(
        _topk_body,
        grid=(NCH,),
        in_specs=[
            pl.BlockSpec((C, D), lambda c: (c, 0)),
            pl.BlockSpec((D, B), lambda c: (0, 0)),
            pl.BlockSpec((1, 1, C), lambda c: (c, 0, 0)),
            pl.BlockSpec((1, B), lambda c: (0, 0)),
        ],
        out_specs=pl.BlockSpec((K, B), lambda c: (0, 0)),
        out_shape=jax.ShapeDtypeStruct((K, B), jnp.int32),
        scratch_shapes=[
            pltpu.VMEM((K, B), jnp.int32),
            pltpu.VMEM((K, B), jnp.int32),
        ],
    )(train_pad, testT, tr_pad, teT)


def _gather_mean(train_y, idx_flat):
    mesh = plsc.VectorSubcoreMesh(core_axis_name="c", subcore_axis_name="s")

    @functools.partial(
        pl.kernel,
        mesh=mesh,
        out_type=jax.ShapeDtypeStruct((B, D), jnp.float32),
        scratch_types=[
            pltpu.VMEM((IW,), jnp.int32),
            pltpu.VMEM((IW, D), jnp.float32),
            pltpu.VMEM((QW, D), jnp.float32),
            pltpu.SemaphoreType.DMA,
        ],
        compiler_params=pltpu.CompilerParams(use_tc_tiling_on_sc=False),
    )
    def k(y_hbm, idx_hbm, out_hbm, idx_v, rows_v, acc_v, sem):
        wid = lax.axis_index("s") * 2 + lax.axis_index("c")
        base = wid * IW
        pltpu.sync_copy(idx_hbm.at[pl.ds(base, IW)], idx_v)
        pltpu.async_copy(y_hbm.at[idx_v], rows_v, sem).wait()

        def q_body(q, carry):
            acc = rows_v[q * K]
            for j in range(1, K):
                acc = acc + rows_v[q * K + j]
            acc_v[q] = acc * (1.0 / K)
            return carry

        lax.fori_loop(0, QW, q_body, 0)
        pltpu.sync_copy(acc_v, out_hbm.at[pl.ds(wid * QW, QW)])

    return k(train_y, idx_flat)


def kernel(train_x, train_y, test_x):
    train_pad = jnp.pad(train_x, ((0, NP - N), (0, 0)))
    testT = test_x.T
    tr = jnp.sum(train_x ** 2, axis=1, keepdims=True)
    te = jnp.sum(test_x ** 2, axis=1, keepdims=True)
    tr_pad = jnp.pad(tr[:, 0], (0, NP - N),
                     constant_values=jnp.inf).reshape(NCH, 1, C)
    idx = _topk_call(train_pad, testT, tr_pad, te.T)
    return _gather_mean(train_y, idx.T.reshape(B * K))
